# Initial kernel scaffold; baseline (speedup 1.0000x reference)
import jax
import jax.numpy as jnp
from jax.experimental import pallas as pl


def kernel(c1_w, c1_b, c2_w, c2_b, f1_w, f1_b, f2_w, f2_b, x):
    raise NotImplementedError("write your pallas kernel here")


# speedup vs baseline: 1.0712x; 1.0712x over previous
import jax
import jax.numpy as jnp
from jax.experimental import pallas as pl
from jax.experimental.pallas import tpu as pltpu


def _fused_net_kernel(x_ref, w1_ref, b1_ref, w2_ref, b2_ref,
                      f1_ref, f1b_ref, f2_ref, f2b_ref, o_ref):
    bn, H, W, _ = x_ref.shape
    C1 = w1_ref.shape[1]
    C2 = w2_ref.shape[1]
    H1, W1 = H - 2, W - 2
    H2, W2 = H1 - 2, W1 - 2
    PH, PW = H2 // 2, W2 // 2

    acc1 = x_ref[:, 0:H1, 0:W1, :] * w1_ref[0:1, :]
    for k in range(1, 9):
        dh, dw = divmod(k, 3)
        acc1 = acc1 + x_ref[:, dh:dh + H1, dw:dw + W1, :] * w1_ref[k:k + 1, :]
    h = jnp.maximum(acc1 + b1_ref[...], 0.0).astype(jnp.bfloat16)

    taps = [h[:, dh:dh + H2, dw:dw + W2, :]
            for dh in range(3) for dw in range(3)]
    patches = jnp.concatenate(taps, axis=-1).reshape(bn * H2 * W2, 9 * C1)
    acc2 = jnp.dot(patches, w2_ref[...],
                   preferred_element_type=jnp.float32)

    pw = acc2.reshape(bn, H2, PW, 2, C2).max(axis=3)
    phw = pw.reshape(bn, PH, 2, PW, C2).max(axis=2)
    feat = jnp.maximum(phw + b2_ref[...], 0.0).astype(jnp.bfloat16)
    feat = feat.reshape(bn, PH * PW * C2)

    hid = jnp.dot(feat, f1_ref[...],
                  preferred_element_type=jnp.float32) + f1b_ref[...]
    hid = jnp.maximum(hid, 0.0).astype(jnp.bfloat16)
    logits = jnp.dot(hid, f2_ref[...],
                     preferred_element_type=jnp.float32) + f2b_ref[...]
    m = jnp.max(logits, axis=-1, keepdims=True)
    shifted = logits - m
    lse = jnp.log(jnp.sum(jnp.exp(shifted), axis=-1, keepdims=True))
    o_ref[...] = (shifted - lse).astype(o_ref.dtype)


def kernel(c1_w, c1_b, c2_w, c2_b, f1_w, f1_b, f2_w, f2_b, x):
    N = x.shape[0]
    x = x.reshape(N, 28, 28, 1)
    C1 = c1_w.shape[1]
    C2 = c2_w.shape[1]
    HID = f1_w.shape[1]
    NOUT = f2_w.shape[1]

    w2 = c2_w.astype(jnp.bfloat16)
    f1 = f1_w.astype(jnp.bfloat16)
    f2 = f2_w.astype(jnp.bfloat16)

    bn = 8
    grid = (N // bn,)
    return pl.pallas_call(
        _fused_net_kernel,
        out_shape=jax.ShapeDtypeStruct((N, NOUT), x.dtype),
        grid=grid,
        in_specs=[
            pl.BlockSpec((bn, 28, 28, 1), lambda n: (n, 0, 0, 0)),
            pl.BlockSpec((9, C1), lambda n: (0, 0)),
            pl.BlockSpec((1, C1), lambda n: (0, 0)),
            pl.BlockSpec((9 * C1, C2), lambda n: (0, 0)),
            pl.BlockSpec((1, C2), lambda n: (0, 0)),
            pl.BlockSpec((f1_w.shape[0], HID), lambda n: (0, 0)),
            pl.BlockSpec((1, HID), lambda n: (0, 0)),
            pl.BlockSpec((HID, NOUT), lambda n: (0, 0)),
            pl.BlockSpec((1, NOUT), lambda n: (0, 0)),
        ],
        out_specs=pl.BlockSpec((bn, NOUT), lambda n: (n, 0)),
        compiler_params=pltpu.CompilerParams(
            dimension_semantics=("parallel",),
            vmem_limit_bytes=64 * 1024 * 1024),
    )(x, c1_w, c1_b, w2, c2_b, f1, f1_b, f2, f2_b)


# speedup vs baseline: 2.5542x; 2.3846x over previous
import jax
import jax.numpy as jnp
from jax.experimental import pallas as pl
from jax.experimental.pallas import tpu as pltpu


def _fused_net_kernel(x_ref, B1_ref, b1_ref, B2_ref, b2_ref,
                      f1_ref, f1b_ref, f2_ref, f2b_ref, o_ref):
    bn = x_ref.shape[0]

    xb = x_ref[...].reshape(bn, 28, 28)
    xc = jnp.concatenate([xb[:, dh:dh + 26, :] for dh in range(3)], axis=-1)
    xc = xc.reshape(bn * 26, 84)
    h1 = jnp.dot(xc, B1_ref[...], preferred_element_type=jnp.float32)
    h1 = jnp.maximum(h1 + b1_ref[...], 0.0).astype(jnp.bfloat16)
    h1 = h1.reshape(bn, 26, 832)

    acc = jnp.dot(h1[:, 0:24, :].reshape(bn * 24, 832), B2_ref[0],
                  preferred_element_type=jnp.float32)
    acc = acc + jnp.dot(h1[:, 1:25, :].reshape(bn * 24, 832), B2_ref[1],
                        preferred_element_type=jnp.float32)
    acc = acc + jnp.dot(h1[:, 2:26, :].reshape(bn * 24, 832), B2_ref[2],
                        preferred_element_type=jnp.float32)

    pw = acc.reshape(bn * 24, 12, 2, 64).max(axis=2)
    pw = pw.reshape(bn, 24, 768)
    ph = pw.reshape(bn, 12, 2, 768).max(axis=2)
    feat = jnp.maximum(ph + b2_ref[...], 0.0).astype(jnp.bfloat16)
    feat = feat.reshape(bn, 9216)

    hid = jnp.dot(feat, f1_ref[...],
                  preferred_element_type=jnp.float32) + f1b_ref[...]
    hid = jnp.maximum(hid, 0.0).astype(jnp.bfloat16)
    logits = jnp.dot(hid, f2_ref[...],
                     preferred_element_type=jnp.float32) + f2b_ref[...]
    m = jnp.max(logits, axis=-1, keepdims=True)
    shifted = logits - m
    lse = jnp.log(jnp.sum(jnp.exp(shifted), axis=-1, keepdims=True))
    o_ref[...] = (shifted - lse).astype(o_ref.dtype)


def kernel(c1_w, c1_b, c2_w, c2_b, f1_w, f1_b, f2_w, f2_b, x):
    N = x.shape[0]
    x2 = x.reshape(N, 784)

    E1 = jnp.stack([jnp.eye(28, 26, k=-t, dtype=jnp.float32)
                    for t in range(3)])
    B1 = jnp.einsum('twv,dtc->dwvc', E1, c1_w.reshape(3, 3, 32))
    B1 = B1.reshape(84, 832)
    E2 = jnp.stack([jnp.eye(26, 24, k=-t, dtype=jnp.float32)
                    for t in range(3)])
    B2 = jnp.einsum('twv,dtcu->dwcvu', E2, c2_w.reshape(3, 3, 32, 64))
    B2 = B2.reshape(3, 832, 1536).astype(jnp.bfloat16)
    b1t = jnp.tile(c1_b, (1, 26))
    b2t = jnp.tile(c2_b, (1, 12))
    f1 = f1_w.astype(jnp.bfloat16)
    f2 = f2_w.astype(jnp.bfloat16)

    bn = 32
    grid = (N // bn,)
    return pl.pallas_call(
        _fused_net_kernel,
        out_shape=jax.ShapeDtypeStruct((N, 10), x.dtype),
        grid=grid,
        in_specs=[
            pl.BlockSpec((bn, 784), lambda n: (n, 0)),
            pl.BlockSpec((84, 832), lambda n: (0, 0)),
            pl.BlockSpec((1, 832), lambda n: (0, 0)),
            pl.BlockSpec((3, 832, 1536), lambda n: (0, 0, 0)),
            pl.BlockSpec((1, 768), lambda n: (0, 0)),
            pl.BlockSpec((9216, 128), lambda n: (0, 0)),
            pl.BlockSpec((1, 128), lambda n: (0, 0)),
            pl.BlockSpec((128, 10), lambda n: (0, 0)),
            pl.BlockSpec((1, 10), lambda n: (0, 0)),
        ],
        out_specs=pl.BlockSpec((bn, 10), lambda n: (n, 0)),
        compiler_params=pltpu.CompilerParams(
            dimension_semantics=("parallel",),
            vmem_limit_bytes=64 * 1024 * 1024),
    )(x2, B1, b1t, B2, b2t, f1, f1_b, f2, f2_b)


# speedup vs baseline: 4.7554x; 1.8617x over previous
import jax
import jax.numpy as jnp
from jax.experimental import pallas as pl
from jax.experimental.pallas import tpu as pltpu


def _fused_net_kernel(x_ref, B1_ref, b1_ref, B2_ref, b2_ref,
                      f1_ref, f1b_ref, f2_ref, f2b_ref, o_ref):
    bn = x_ref.shape[0]

    xb = x_ref[...].reshape(bn, 28, 28)
    xc = jnp.concatenate([xb[:, dh:dh + 26, :] for dh in range(3)], axis=-1)
    xc = xc.reshape(bn * 26, 84)
    h1 = jnp.dot(xc, B1_ref[...], preferred_element_type=jnp.float32)
    h1 = jnp.maximum(h1 + b1_ref[...], 0.0).astype(jnp.bfloat16)
    h1 = h1.reshape(bn, 26, 832)

    h1r = h1.reshape(bn, 13, 2, 832)
    h1e = h1r[:, :, 0, :]
    h1o = h1r[:, :, 1, :]
    slabs = {}
    for p in range(2):
        for dh in range(3):
            src = h1o if (p + dh) % 2 else h1e
            s = (p + dh) // 2
            slabs[(p, dh)] = src[:, s:s + 12, :].reshape(bn * 12, 832)

    def quad(p, w):
        a = jnp.dot(slabs[(p, 0)], B2_ref[0, w],
                    preferred_element_type=jnp.float32)
        a = a + jnp.dot(slabs[(p, 1)], B2_ref[1, w],
                        preferred_element_type=jnp.float32)
        return a + jnp.dot(slabs[(p, 2)], B2_ref[2, w],
                           preferred_element_type=jnp.float32)

    pooled = jnp.maximum(jnp.maximum(quad(0, 0), quad(0, 1)),
                         jnp.maximum(quad(1, 0), quad(1, 1)))
    ph = pooled.reshape(bn, 12, 768)
    feat = jnp.maximum(ph + b2_ref[...], 0.0).astype(jnp.bfloat16)
    feat = feat.reshape(bn, 9216)

    hid = jnp.dot(feat, f1_ref[...],
                  preferred_element_type=jnp.float32) + f1b_ref[...]
    hid = jnp.maximum(hid, 0.0).astype(jnp.bfloat16)
    logits = jnp.dot(hid, f2_ref[...],
                     preferred_element_type=jnp.float32) + f2b_ref[...]
    m = jnp.max(logits, axis=-1, keepdims=True)
    shifted = logits - m
    lse = jnp.log(jnp.sum(jnp.exp(shifted), axis=-1, keepdims=True))
    o_ref[...] = (shifted - lse).astype(o_ref.dtype)


def kernel(c1_w, c1_b, c2_w, c2_b, f1_w, f1_b, f2_w, f2_b, x):
    N = x.shape[0]
    x2 = x.reshape(N, 784)

    E1 = jnp.stack([jnp.eye(28, 26, k=-t, dtype=jnp.float32)
                    for t in range(3)])
    B1 = jnp.einsum('twv,dtc->dwvc', E1, c1_w.reshape(3, 3, 32))
    B1 = B1.reshape(84, 832)
    E2 = jnp.stack([jnp.eye(26, 24, k=-t, dtype=jnp.float32)
                    for t in range(3)])
    B2 = jnp.einsum('twv,dtcu->dwcvu', E2, c2_w.reshape(3, 3, 32, 64))
    B2 = B2.reshape(3, 832, 24, 64)
    B2 = jnp.stack([B2[:, :, 0::2, :].reshape(3, 832, 768),
                    B2[:, :, 1::2, :].reshape(3, 832, 768)], axis=1)
    B2 = B2.astype(jnp.bfloat16)
    b1t = jnp.tile(c1_b, (1, 26))
    b2t = jnp.tile(c2_b, (1, 12))
    f1 = f1_w.astype(jnp.bfloat16)
    f2 = f2_w.astype(jnp.bfloat16)

    bn = 32
    grid = (N // bn,)
    return pl.pallas_call(
        _fused_net_kernel,
        out_shape=jax.ShapeDtypeStruct((N, 10), x.dtype),
        grid=grid,
        in_specs=[
            pl.BlockSpec((bn, 784), lambda n: (n, 0)),
            pl.BlockSpec((84, 832), lambda n: (0, 0)),
            pl.BlockSpec((1, 832), lambda n: (0, 0)),
            pl.BlockSpec((3, 2, 832, 768), lambda n: (0, 0, 0, 0)),
            pl.BlockSpec((1, 768), lambda n: (0, 0)),
            pl.BlockSpec((9216, 128), lambda n: (0, 0)),
            pl.BlockSpec((1, 128), lambda n: (0, 0)),
            pl.BlockSpec((128, 10), lambda n: (0, 0)),
            pl.BlockSpec((1, 10), lambda n: (0, 0)),
        ],
        out_specs=pl.BlockSpec((bn, 10), lambda n: (n, 0)),
        compiler_params=pltpu.CompilerParams(
            dimension_semantics=("parallel",),
            vmem_limit_bytes=64 * 1024 * 1024),
    )(x2, B1, b1t, B2, b2t, f1, f1_b, f2, f2_b)


# speedup vs baseline: 5.9515x; 1.2515x over previous
import jax
import jax.numpy as jnp
from jax.experimental import pallas as pl
from jax.experimental.pallas import tpu as pltpu


def _fused_net_kernel(xe_ref, xo_ref, B1_ref, b1_ref, B2_ref, b2_ref,
                      f1_ref, f1b_ref, f2_ref, f2b_ref, o_ref):
    bn = xe_ref.shape[0]

    xe = xe_ref[...].reshape(bn, 14, 28)
    xo = xo_ref[...].reshape(bn, 14, 28)
    xce = jnp.concatenate([xe[:, 0:13, :], xo[:, 0:13, :], xe[:, 1:14, :]],
                          axis=-1).reshape(bn * 13, 84).astype(jnp.bfloat16)
    xco = jnp.concatenate([xo[:, 0:13, :], xe[:, 1:14, :], xo[:, 1:14, :]],
                          axis=-1).reshape(bn * 13, 84).astype(jnp.bfloat16)
    h1e = jnp.maximum(jnp.dot(xce, B1_ref[...],
                              preferred_element_type=jnp.float32)
                      + b1_ref[...], 0.0).astype(jnp.bfloat16)
    h1o = jnp.maximum(jnp.dot(xco, B1_ref[...],
                              preferred_element_type=jnp.float32)
                      + b1_ref[...], 0.0).astype(jnp.bfloat16)
    h1e = h1e.reshape(bn, 13, 832)
    h1o = h1o.reshape(bn, 13, 832)

    slabs = {}
    for p in range(2):
        for dh in range(3):
            src = h1o if (p + dh) % 2 else h1e
            s = (p + dh) // 2
            slabs[(p, dh)] = src[:, s:s + 12, :].reshape(bn * 12, 832)

    def quad(p, w):
        a = jnp.dot(slabs[(p, 0)], B2_ref[0, w],
                    preferred_element_type=jnp.float32)
        a = a + jnp.dot(slabs[(p, 1)], B2_ref[1, w],
                        preferred_element_type=jnp.float32)
        return a + jnp.dot(slabs[(p, 2)], B2_ref[2, w],
                           preferred_element_type=jnp.float32)

    pooled = jnp.maximum(jnp.maximum(quad(0, 0), quad(0, 1)),
                         jnp.maximum(quad(1, 0), quad(1, 1)))
    ph = pooled.reshape(bn, 12, 768)
    feat = jnp.maximum(ph + b2_ref[...], 0.0).astype(jnp.bfloat16)
    feat = feat.reshape(bn, 9216)

    hid = jnp.dot(feat, f1_ref[...],
                  preferred_element_type=jnp.float32) + f1b_ref[...]
    hid = jnp.maximum(hid, 0.0).astype(jnp.bfloat16)
    logits = jnp.dot(hid, f2_ref[...],
                     preferred_element_type=jnp.float32) + f2b_ref[...]
    m = jnp.max(logits, axis=-1, keepdims=True)
    shifted = logits - m
    lse = jnp.log(jnp.sum(jnp.exp(shifted), axis=-1, keepdims=True))
    o_ref[...] = (shifted - lse).astype(o_ref.dtype)


def kernel(c1_w, c1_b, c2_w, c2_b, f1_w, f1_b, f2_w, f2_b, x):
    N = x.shape[0]
    xr = x.reshape(N, 28, 28)
    xe = xr[:, 0::2, :].reshape(N, 392)
    xo = xr[:, 1::2, :].reshape(N, 392)

    E1 = jnp.stack([jnp.eye(28, 26, k=-t, dtype=jnp.float32)
                    for t in range(3)])
    B1 = jnp.einsum('twv,dtc->dwvc', E1, c1_w.reshape(3, 3, 32))
    B1 = B1.reshape(84, 832).astype(jnp.bfloat16)
    E2 = jnp.stack([jnp.eye(26, 24, k=-t, dtype=jnp.float32)
                    for t in range(3)])
    B2 = jnp.einsum('twv,dtcu->dwcvu', E2, c2_w.reshape(3, 3, 32, 64))
    B2 = B2.reshape(3, 832, 24, 64)
    B2 = jnp.stack([B2[:, :, 0::2, :].reshape(3, 832, 768),
                    B2[:, :, 1::2, :].reshape(3, 832, 768)], axis=1)
    B2 = B2.astype(jnp.bfloat16)
    b1t = jnp.tile(c1_b, (1, 26))
    b2t = jnp.tile(c2_b, (1, 12))
    f1 = f1_w.astype(jnp.bfloat16)
    f2 = f2_w.astype(jnp.bfloat16)

    bn = 32
    grid = (N // bn,)
    return pl.pallas_call(
        _fused_net_kernel,
        out_shape=jax.ShapeDtypeStruct((N, 10), x.dtype),
        grid=grid,
        in_specs=[
            pl.BlockSpec((bn, 392), lambda n: (n, 0)),
            pl.BlockSpec((bn, 392), lambda n: (n, 0)),
            pl.BlockSpec((84, 832), lambda n: (0, 0)),
            pl.BlockSpec((1, 832), lambda n: (0, 0)),
            pl.BlockSpec((3, 2, 832, 768), lambda n: (0, 0, 0, 0)),
            pl.BlockSpec((1, 768), lambda n: (0, 0)),
            pl.BlockSpec((9216, 128), lambda n: (0, 0)),
            pl.BlockSpec((1, 128), lambda n: (0, 0)),
            pl.BlockSpec((128, 10), lambda n: (0, 0)),
            pl.BlockSpec((1, 10), lambda n: (0, 0)),
        ],
        out_specs=pl.BlockSpec((bn, 10), lambda n: (n, 0)),
        compiler_params=pltpu.CompilerParams(
            dimension_semantics=("parallel",),
            vmem_limit_bytes=64 * 1024 * 1024),
    )(xe, xo, B1, b1t, B2, b2t, f1, f1_b, f2, f2_b)


# speedup vs baseline: 6.2868x; 1.0563x over previous
import jax
import jax.numpy as jnp
from jax.experimental import pallas as pl
from jax.experimental.pallas import tpu as pltpu


def _fused_net_kernel(xe_ref, xo_ref, B1_ref, b1_ref, B2_ref, b2_ref,
                      f1_ref, f1b_ref, f2_ref, f2b_ref, o_ref):
    bn = xe_ref.shape[0]

    xe = xe_ref[...].reshape(bn, 14, 28)
    xo = xo_ref[...].reshape(bn, 14, 28)
    xce = jnp.concatenate([xe[:, 0:13, :], xo[:, 0:13, :], xe[:, 1:14, :]],
                          axis=-1).reshape(bn * 13, 84).astype(jnp.bfloat16)
    xco = jnp.concatenate([xo[:, 0:13, :], xe[:, 1:14, :], xo[:, 1:14, :]],
                          axis=-1).reshape(bn * 13, 84).astype(jnp.bfloat16)
    h1e = jnp.maximum(jnp.dot(xce, B1_ref[...],
                              preferred_element_type=jnp.float32)
                      + b1_ref[...], 0.0).astype(jnp.bfloat16)
    h1o = jnp.maximum(jnp.dot(xco, B1_ref[...],
                              preferred_element_type=jnp.float32)
                      + b1_ref[...], 0.0).astype(jnp.bfloat16)
    h1e = h1e.reshape(bn, 13, 832)
    h1o = h1o.reshape(bn, 13, 832)

    slabs = {}
    for p in range(2):
        for dh in range(3):
            src = h1o if (p + dh) % 2 else h1e
            s = (p + dh) // 2
            slabs[(p, dh)] = src[:, s:s + 12, :].reshape(bn * 12, 832)

    def quad(p, w):
        a = jnp.dot(slabs[(p, 0)], B2_ref[0, w],
                    preferred_element_type=jnp.float32)
        a = a + jnp.dot(slabs[(p, 1)], B2_ref[1, w],
                        preferred_element_type=jnp.float32)
        return a + jnp.dot(slabs[(p, 2)], B2_ref[2, w],
                           preferred_element_type=jnp.float32)

    pooled = jnp.maximum(jnp.maximum(quad(0, 0), quad(0, 1)),
                         jnp.maximum(quad(1, 0), quad(1, 1)))
    ph = pooled.reshape(bn, 12, 768)
    feat = jnp.maximum(ph + b2_ref[...], 0.0).astype(jnp.bfloat16)
    feat = feat.reshape(bn, 9216)

    hid = jnp.dot(feat, f1_ref[...],
                  preferred_element_type=jnp.float32) + f1b_ref[...]
    hid = jnp.maximum(hid, 0.0).astype(jnp.bfloat16)
    logits = jnp.dot(hid, f2_ref[...],
                     preferred_element_type=jnp.float32) + f2b_ref[...]
    m = jnp.max(logits, axis=-1, keepdims=True)
    shifted = logits - m
    lse = jnp.log(jnp.sum(jnp.exp(shifted), axis=-1, keepdims=True))
    o_ref[...] = (shifted - lse).astype(o_ref.dtype)


def kernel(c1_w, c1_b, c2_w, c2_b, f1_w, f1_b, f2_w, f2_b, x):
    N = x.shape[0]
    xr = x.reshape(N, 28, 28)
    xe = xr[:, 0::2, :].reshape(N, 392)
    xo = xr[:, 1::2, :].reshape(N, 392)

    E1 = jnp.stack([jnp.eye(28, 26, k=-t, dtype=jnp.float32)
                    for t in range(3)])
    B1 = jnp.einsum('twv,dtc->dwvc', E1, c1_w.reshape(3, 3, 32))
    B1 = B1.reshape(84, 832).astype(jnp.bfloat16)
    E2 = jnp.stack([jnp.eye(26, 24, k=-t, dtype=jnp.float32)
                    for t in range(3)])
    B2 = jnp.einsum('twv,dtcu->dwcvu', E2, c2_w.reshape(3, 3, 32, 64))
    B2 = B2.reshape(3, 832, 24, 64)
    B2 = jnp.stack([B2[:, :, 0::2, :].reshape(3, 832, 768),
                    B2[:, :, 1::2, :].reshape(3, 832, 768)], axis=1)
    B2 = B2.astype(jnp.bfloat16)
    b1t = jnp.tile(c1_b, (1, 26))
    b2t = jnp.tile(c2_b, (1, 12))
    f1 = f1_w.astype(jnp.bfloat16)
    f2 = f2_w.astype(jnp.bfloat16)

    bn = 64
    grid = (N // bn,)
    return pl.pallas_call(
        _fused_net_kernel,
        out_shape=jax.ShapeDtypeStruct((N, 10), x.dtype),
        grid=grid,
        in_specs=[
            pl.BlockSpec((bn, 392), lambda n: (n, 0)),
            pl.BlockSpec((bn, 392), lambda n: (n, 0)),
            pl.BlockSpec((84, 832), lambda n: (0, 0)),
            pl.BlockSpec((1, 832), lambda n: (0, 0)),
            pl.BlockSpec((3, 2, 832, 768), lambda n: (0, 0, 0, 0)),
            pl.BlockSpec((1, 768), lambda n: (0, 0)),
            pl.BlockSpec((9216, 128), lambda n: (0, 0)),
            pl.BlockSpec((1, 128), lambda n: (0, 0)),
            pl.BlockSpec((128, 10), lambda n: (0, 0)),
            pl.BlockSpec((1, 10), lambda n: (0, 0)),
        ],
        out_specs=pl.BlockSpec((bn, 10), lambda n: (n, 0)),
        compiler_params=pltpu.CompilerParams(
            dimension_semantics=("parallel",),
            vmem_limit_bytes=64 * 1024 * 1024),
    )(xe, xo, B1, b1t, B2, b2t, f1, f1_b, f2, f2_b)


# speedup vs baseline: 10.2141x; 1.6247x over previous
import jax
import jax.numpy as jnp
from jax.experimental import pallas as pl
from jax.experimental.pallas import tpu as pltpu


def _fused_net_kernel(xe_ref, xo_ref, B1_ref, b1_ref, B2_ref, b2_ref,
                      f1_ref, f1b_ref, f2_ref, f2b_ref, o_ref):
    bn = xe_ref.shape[0]

    xe = xe_ref[...].reshape(bn, 14, 28)
    xo = xo_ref[...].reshape(bn, 14, 28)
    xce = jnp.concatenate([xe[:, 0:13, :], xo[:, 0:13, :], xe[:, 1:14, :]],
                          axis=-1).reshape(bn * 13, 84).astype(jnp.bfloat16)
    xco = jnp.concatenate([xo[:, 0:13, :], xe[:, 1:14, :], xo[:, 1:14, :]],
                          axis=-1).reshape(bn * 13, 84).astype(jnp.bfloat16)
    h1e = jnp.maximum(jnp.dot(xce, B1_ref[...],
                              preferred_element_type=jnp.float32)
                      + b1_ref[...], 0.0).astype(jnp.bfloat16)
    h1o = jnp.maximum(jnp.dot(xco, B1_ref[...],
                              preferred_element_type=jnp.float32)
                      + b1_ref[...], 0.0).astype(jnp.bfloat16)
    h1e = h1e.reshape(bn, 13, 1024)
    h1o = h1o.reshape(bn, 13, 1024)

    slabs = {}
    for p in range(2):
        for dh in range(3):
            src = h1o if (p + dh) % 2 else h1e
            s = (p + dh) // 2
            slabs[(p, dh)] = src[:, s:s + 12, :].reshape(bn * 12, 1024)

    def row_block(p):
        halves = []
        for h in range(2):
            a = jnp.dot(slabs[(p, 0)][:, 512 * h:512 * h + 512], B2_ref[0, h],
                        preferred_element_type=jnp.float32)
            a = a + jnp.dot(slabs[(p, 1)][:, 512 * h:512 * h + 512],
                            B2_ref[1, h],
                            preferred_element_type=jnp.float32)
            a = a + jnp.dot(slabs[(p, 2)][:, 512 * h:512 * h + 512],
                            B2_ref[2, h],
                            preferred_element_type=jnp.float32)
            halves.append(jnp.maximum(a[:, 0:384], a[:, 384:768]))
        return jnp.concatenate(halves, axis=-1)

    pooled = jnp.maximum(row_block(0), row_block(1))
    ph = pooled.reshape(bn, 12, 768)
    feat = jnp.maximum(ph + b2_ref[...], 0.0).astype(jnp.bfloat16)
    feat = feat.reshape(bn, 9216)

    hid = jnp.dot(feat, f1_ref[...],
                  preferred_element_type=jnp.float32) + f1b_ref[...]
    hid = jnp.maximum(hid, 0.0).astype(jnp.bfloat16)
    logits = jnp.dot(hid, f2_ref[...],
                     preferred_element_type=jnp.float32) + f2b_ref[...]
    m = jnp.max(logits, axis=-1, keepdims=True)
    shifted = logits - m
    lse = jnp.log(jnp.sum(jnp.exp(shifted), axis=-1, keepdims=True))
    o_ref[...] = (shifted - lse).astype(o_ref.dtype)


def kernel(c1_w, c1_b, c2_w, c2_b, f1_w, f1_b, f2_w, f2_b, x):
    N = x.shape[0]
    xr = x.reshape(N, 28, 28)
    xe = xr[:, 0::2, :].reshape(N, 392)
    xo = xr[:, 1::2, :].reshape(N, 392)

    E1 = jnp.stack([jnp.eye(28, 26, k=-t, dtype=jnp.float32)
                    for t in range(3)])
    B1 = jnp.einsum('twv,dtc->dwvc', E1, c1_w.reshape(3, 3, 32))
    B1 = B1.reshape(84, 26, 32)
    pos = jnp.arange(32)
    wi_of_pos = jnp.where(pos >= 16, pos - 4, pos)
    valid = (wi_of_pos <= 25).astype(jnp.float32)
    B1x = B1[:, jnp.clip(wi_of_pos, 0, 25), :] * valid[None, :, None]
    B1x = B1x.reshape(84, 1024).astype(jnp.bfloat16)

    pl_i = jnp.arange(16)[:, None, None, None]
    wpl_i = jnp.arange(6)[None, :, None, None]
    wpar_i = jnp.arange(2)[None, None, :, None]
    dw_i = jnp.arange(3)[None, None, None, :]
    D = (pl_i == 2 * wpl_i + wpar_i + dw_i).astype(jnp.float32)
    w2r = c2_w.reshape(3, 3, 32, 64)
    B2c = jnp.einsum('pwqd,hdcu->hpcqwu', D, w2r)
    B2c = B2c.reshape(3, 512, 768)
    rmask = (jnp.arange(512) < 448).astype(jnp.float32)[None, :, None]
    B2x = jnp.stack([B2c, B2c * rmask], axis=1)
    B2x = B2x.astype(jnp.bfloat16)

    b1t = jnp.tile(c1_b, (1, 32))
    b2t = jnp.tile(c2_b, (1, 12))
    f1 = f1_w.astype(jnp.bfloat16)
    f2 = f2_w.astype(jnp.bfloat16)

    bn = 64
    grid = (N // bn,)
    return pl.pallas_call(
        _fused_net_kernel,
        out_shape=jax.ShapeDtypeStruct((N, 10), x.dtype),
        grid=grid,
        in_specs=[
            pl.BlockSpec((bn, 392), lambda n: (n, 0)),
            pl.BlockSpec((bn, 392), lambda n: (n, 0)),
            pl.BlockSpec((84, 1024), lambda n: (0, 0)),
            pl.BlockSpec((1, 1024), lambda n: (0, 0)),
            pl.BlockSpec((3, 2, 512, 768), lambda n: (0, 0, 0, 0)),
            pl.BlockSpec((1, 768), lambda n: (0, 0)),
            pl.BlockSpec((9216, 128), lambda n: (0, 0)),
            pl.BlockSpec((1, 128), lambda n: (0, 0)),
            pl.BlockSpec((128, 10), lambda n: (0, 0)),
            pl.BlockSpec((1, 10), lambda n: (0, 0)),
        ],
        out_specs=pl.BlockSpec((bn, 10), lambda n: (n, 0)),
        compiler_params=pltpu.CompilerParams(
            dimension_semantics=("parallel",),
            vmem_limit_bytes=64 * 1024 * 1024),
    )(xe, xo, B1x, b1t, B2x, b2t, f1, f1_b, f2, f2_b)


# speedup vs baseline: 11.1115x; 1.0879x over previous
import jax
import jax.numpy as jnp
from jax.experimental import pallas as pl
from jax.experimental.pallas import tpu as pltpu


def _fused_net_kernel(xe_ref, xo_ref, B1_ref, B2_ref, b2_ref,
                      f1_ref, f1b_ref, f2_ref, f2b_ref, o_ref):
    bn = xe_ref.shape[0]

    xe = xe_ref[...].reshape(bn, 14, 28)
    xo = xo_ref[...].reshape(bn, 14, 28)
    ones = jnp.ones((bn, 13, 1), jnp.float32)
    xce = jnp.concatenate([xe[:, 0:13, :], xo[:, 0:13, :], xe[:, 1:14, :],
                           ones],
                          axis=-1).reshape(bn * 13, 85).astype(jnp.bfloat16)
    xco = jnp.concatenate([xo[:, 0:13, :], xe[:, 1:14, :], xo[:, 1:14, :],
                           ones],
                          axis=-1).reshape(bn * 13, 85).astype(jnp.bfloat16)
    h1e = jnp.maximum(jnp.dot(xce, B1_ref[...],
                              preferred_element_type=jnp.float32),
                      0.0).astype(jnp.bfloat16)
    h1o = jnp.maximum(jnp.dot(xco, B1_ref[...],
                              preferred_element_type=jnp.float32),
                      0.0).astype(jnp.bfloat16)
    h1e = h1e.reshape(bn, 13, 1024)
    h1o = h1o.reshape(bn, 13, 1024)

    slabs = {}
    for p in range(2):
        for dh in range(3):
            src = h1o if (p + dh) % 2 else h1e
            s = (p + dh) // 2
            slabs[(p, dh)] = src[:, s:s + 12, :].reshape(bn * 12, 1024)

    halves = []
    for h in range(2):
        lhs = jnp.concatenate(
            [jnp.concatenate([slabs[(p, dh)][:, 512 * h:512 * h + 512]
                              for dh in range(3)], axis=1)
             for p in range(2)], axis=0)
        a = jnp.dot(lhs, B2_ref[h], preferred_element_type=jnp.float32)
        aw = jnp.maximum(a[:, 0:384], a[:, 384:768])
        halves.append(jnp.maximum(aw[0:bn * 12], aw[bn * 12:]))
    pooled = jnp.concatenate(halves, axis=-1)
    ph = pooled.reshape(bn, 12, 768)
    feat = jnp.maximum(ph + b2_ref[...], 0.0).astype(jnp.bfloat16)
    feat = feat.reshape(bn, 9216)

    hid = jnp.dot(feat, f1_ref[...],
                  preferred_element_type=jnp.float32) + f1b_ref[...]
    hid = jnp.maximum(hid, 0.0).astype(jnp.bfloat16)
    logits = jnp.dot(hid, f2_ref[...],
                     preferred_element_type=jnp.float32) + f2b_ref[...]
    m = jnp.max(logits, axis=-1, keepdims=True)
    shifted = logits - m
    lse = jnp.log(jnp.sum(jnp.exp(shifted), axis=-1, keepdims=True))
    o_ref[...] = (shifted - lse).astype(o_ref.dtype)


def kernel(c1_w, c1_b, c2_w, c2_b, f1_w, f1_b, f2_w, f2_b, x):
    N = x.shape[0]
    xr = x.reshape(N, 28, 28)
    xe = xr[:, 0::2, :].reshape(N, 392)
    xo = xr[:, 1::2, :].reshape(N, 392)

    E1 = jnp.stack([jnp.eye(28, 26, k=-t, dtype=jnp.float32)
                    for t in range(3)])
    B1 = jnp.einsum('twv,dtc->dwvc', E1, c1_w.reshape(3, 3, 32))
    B1 = B1.reshape(84, 26, 32)
    pos = jnp.arange(32)
    wi_of_pos = jnp.where(pos >= 16, pos - 4, pos)
    valid = (wi_of_pos <= 25).astype(jnp.float32)
    B1x = B1[:, jnp.clip(wi_of_pos, 0, 25), :] * valid[None, :, None]
    B1x = B1x.reshape(84, 1024)
    b1row = jnp.tile(c1_b, (1, 32))
    B1x = jnp.concatenate([B1x, b1row], axis=0).astype(jnp.bfloat16)

    pl_i = jnp.arange(16)[:, None, None, None]
    wpl_i = jnp.arange(6)[None, :, None, None]
    wpar_i = jnp.arange(2)[None, None, :, None]
    dw_i = jnp.arange(3)[None, None, None, :]
    D = (pl_i == 2 * wpl_i + wpar_i + dw_i).astype(jnp.float32)
    w2r = c2_w.reshape(3, 3, 32, 64)
    B2c = jnp.einsum('pwqd,hdcu->hpcqwu', D, w2r)
    B2c = B2c.reshape(3, 512, 768)
    rmask = (jnp.arange(512) < 448).astype(jnp.float32)[None, :, None]
    B2x = jnp.stack([B2c, B2c * rmask], axis=1)
    B2x = B2x.transpose(1, 0, 2, 3).reshape(2, 1536, 768).astype(jnp.bfloat16)

    b2t = jnp.tile(c2_b, (1, 12))
    f1 = f1_w.astype(jnp.bfloat16)
    f2 = f2_w.astype(jnp.bfloat16)

    bn = 64
    grid = (N // bn,)
    return pl.pallas_call(
        _fused_net_kernel,
        out_shape=jax.ShapeDtypeStruct((N, 10), x.dtype),
        grid=grid,
        in_specs=[
            pl.BlockSpec((bn, 392), lambda n: (n, 0)),
            pl.BlockSpec((bn, 392), lambda n: (n, 0)),
            pl.BlockSpec((85, 1024), lambda n: (0, 0)),
            pl.BlockSpec((2, 1536, 768), lambda n: (0, 0, 0)),
            pl.BlockSpec((1, 768), lambda n: (0, 0)),
            pl.BlockSpec((9216, 128), lambda n: (0, 0)),
            pl.BlockSpec((1, 128), lambda n: (0, 0)),
            pl.BlockSpec((128, 10), lambda n: (0, 0)),
            pl.BlockSpec((1, 10), lambda n: (0, 0)),
        ],
        out_specs=pl.BlockSpec((bn, 10), lambda n: (n, 0)),
        compiler_params=pltpu.CompilerParams(
            dimension_semantics=("parallel",),
            vmem_limit_bytes=64 * 1024 * 1024),
    )(xe, xo, B1x, B2x, b2t, f1, f1_b, f2, f2_b)


# speedup vs baseline: 13.4145x; 1.2073x over previous
import jax
import jax.numpy as jnp
from jax.experimental import pallas as pl
from jax.experimental.pallas import tpu as pltpu


def _fused_net_kernel(xe_ref, xo_ref, B1_ref, B2_ref, b2_ref,
                      f1_ref, f1b_ref, f2_ref, f2b_ref, o_ref):
    bn = xe_ref.shape[1]

    xe = xe_ref[...]
    xo = xo_ref[...]
    ones = jnp.ones((13, bn, 1), jnp.float32)
    xce = jnp.concatenate([xe[0:13], xo[0:13], xe[1:14], ones],
                          axis=-1).reshape(13 * bn, 85).astype(jnp.bfloat16)
    xco = jnp.concatenate([xo[0:13], xe[1:14], xo[1:14], ones],
                          axis=-1).reshape(13 * bn, 85).astype(jnp.bfloat16)
    h1e = jnp.maximum(jnp.dot(xce, B1_ref[...],
                              preferred_element_type=jnp.float32),
                      0.0).astype(jnp.bfloat16)
    h1o = jnp.maximum(jnp.dot(xco, B1_ref[...],
                              preferred_element_type=jnp.float32),
                      0.0).astype(jnp.bfloat16)

    slabs = {}
    for p in range(2):
        for dh in range(3):
            src = h1o if (p + dh) % 2 else h1e
            s = (p + dh) // 2
            slabs[(p, dh)] = src[s * bn:(s + 12) * bn]

    halves = []
    for h in range(2):
        lhs = jnp.concatenate(
            [jnp.concatenate([slabs[(p, dh)][:, 512 * h:512 * h + 512]
                              for dh in range(3)], axis=1)
             for p in range(2)], axis=0)
        a = jnp.dot(lhs, B2_ref[h], preferred_element_type=jnp.float32)
        aw = jnp.maximum(a[:, 0:384], a[:, 384:768])
        halves.append(jnp.maximum(aw[0:12 * bn], aw[12 * bn:]))
    pooled = jnp.concatenate(halves, axis=-1)
    feat = jnp.maximum(pooled + b2_ref[...], 0.0).astype(jnp.bfloat16)

    hid = jnp.dot(feat[0:bn], f1_ref[0],
                  preferred_element_type=jnp.float32)
    for k in range(1, 12):
        hid = hid + jnp.dot(feat[k * bn:(k + 1) * bn], f1_ref[k],
                            preferred_element_type=jnp.float32)
    hid = jnp.maximum(hid + f1b_ref[...], 0.0).astype(jnp.bfloat16)
    logits = jnp.dot(hid, f2_ref[...],
                     preferred_element_type=jnp.float32) + f2b_ref[...]
    m = jnp.max(logits, axis=-1, keepdims=True)
    shifted = logits - m
    lse = jnp.log(jnp.sum(jnp.exp(shifted), axis=-1, keepdims=True))
    o_ref[...] = (shifted - lse).astype(o_ref.dtype)


def kernel(c1_w, c1_b, c2_w, c2_b, f1_w, f1_b, f2_w, f2_b, x):
    N = x.shape[0]
    xr = x.reshape(N, 28, 28)
    xe = xr[:, 0::2, :].transpose(1, 0, 2)
    xo = xr[:, 1::2, :].transpose(1, 0, 2)

    E1 = jnp.stack([jnp.eye(28, 26, k=-t, dtype=jnp.float32)
                    for t in range(3)])
    B1 = jnp.einsum('twv,dtc->dwvc', E1, c1_w.reshape(3, 3, 32))
    B1 = B1.reshape(84, 26, 32)
    pos = jnp.arange(32)
    wi_of_pos = jnp.where(pos >= 16, pos - 4, pos)
    valid = (wi_of_pos <= 25).astype(jnp.float32)
    B1x = B1[:, jnp.clip(wi_of_pos, 0, 25), :] * valid[None, :, None]
    B1x = B1x.reshape(84, 1024)
    b1row = jnp.tile(c1_b, (1, 32))
    B1x = jnp.concatenate([B1x, b1row], axis=0).astype(jnp.bfloat16)

    pl_i = jnp.arange(16)[:, None, None, None]
    wpl_i = jnp.arange(6)[None, :, None, None]
    wpar_i = jnp.arange(2)[None, None, :, None]
    dw_i = jnp.arange(3)[None, None, None, :]
    D = (pl_i == 2 * wpl_i + wpar_i + dw_i).astype(jnp.float32)
    w2r = c2_w.reshape(3, 3, 32, 64)
    B2c = jnp.einsum('pwqd,hdcu->hpcqwu', D, w2r)
    B2c = B2c.reshape(3, 512, 768)
    rmask = (jnp.arange(512) < 448).astype(jnp.float32)[None, :, None]
    B2x = jnp.stack([B2c, B2c * rmask], axis=1)
    B2x = B2x.transpose(1, 0, 2, 3).reshape(2, 1536, 768).astype(jnp.bfloat16)

    b2t = jnp.tile(c2_b, (1, 12))
    f1 = f1_w.astype(jnp.bfloat16).reshape(12, 768, 128)
    f2 = f2_w.astype(jnp.bfloat16)

    bn = 64
    grid = (N // bn,)
    return pl.pallas_call(
        _fused_net_kernel,
        out_shape=jax.ShapeDtypeStruct((N, 10), x.dtype),
        grid=grid,
        in_specs=[
            pl.BlockSpec((14, bn, 28), lambda n: (0, n, 0)),
            pl.BlockSpec((14, bn, 28), lambda n: (0, n, 0)),
            pl.BlockSpec((85, 1024), lambda n: (0, 0)),
            pl.BlockSpec((2, 1536, 768), lambda n: (0, 0, 0)),
            pl.BlockSpec((1, 768), lambda n: (0, 0)),
            pl.BlockSpec((12, 768, 128), lambda n: (0, 0, 0)),
            pl.BlockSpec((1, 128), lambda n: (0, 0)),
            pl.BlockSpec((128, 10), lambda n: (0, 0)),
            pl.BlockSpec((1, 10), lambda n: (0, 0)),
        ],
        out_specs=pl.BlockSpec((bn, 10), lambda n: (n, 0)),
        compiler_params=pltpu.CompilerParams(
            dimension_semantics=("parallel",),
            vmem_limit_bytes=64 * 1024 * 1024),
    )(xe, xo, B1x, B2x, b2t, f1, f1_b, f2, f2_b)


# speedup vs baseline: 20.2339x; 1.5084x over previous
import jax
import jax.numpy as jnp
from jax.experimental import pallas as pl
from jax.experimental.pallas import tpu as pltpu


def _fused_net_kernel(xe_ref, xo_ref, B1_ref, B2_ref, b2_ref,
                      f1_ref, f1b_ref, f2_ref, f2b_ref, o_ref):
    bn = xe_ref.shape[1]

    xe = xe_ref[...]
    xo = xo_ref[...]
    ones = jnp.ones((13, bn, 1), jnp.float32)
    xce = jnp.concatenate([xe[0:13], xo[0:13], xe[1:14], ones],
                          axis=-1).reshape(13 * bn, 85).astype(jnp.bfloat16)
    xco = jnp.concatenate([xo[0:13], xe[1:14], xo[1:14], ones],
                          axis=-1).reshape(13 * bn, 85).astype(jnp.bfloat16)
    h1e = jnp.maximum(jnp.dot(xce, B1_ref[...],
                              preferred_element_type=jnp.float32),
                      0.0).astype(jnp.bfloat16)
    h1o = jnp.maximum(jnp.dot(xco, B1_ref[...],
                              preferred_element_type=jnp.float32),
                      0.0).astype(jnp.bfloat16)

    slabs = {}
    for p in range(2):
        for dh in range(3):
            src = h1o if (p + dh) % 2 else h1e
            s = (p + dh) // 2
            slabs[(p, dh)] = src[s * bn:(s + 12) * bn]

    groups = []
    for g in range(6):
        lhs = jnp.concatenate(
            [jnp.concatenate([slabs[(p, dh)][:, 256 * g:256 * g + 256]
                              for dh in range(3)], axis=1)
             for p in range(2)], axis=0)
        a = jnp.dot(lhs, B2_ref[g], preferred_element_type=jnp.float32)
        aw = jnp.maximum(a[:, 0:128], a[:, 128:256])
        groups.append(jnp.maximum(aw[0:12 * bn], aw[12 * bn:]))
    pooled = jnp.concatenate(groups, axis=-1)
    feat = jnp.maximum(pooled + b2_ref[...], 0.0).astype(jnp.bfloat16)

    hid = jnp.dot(feat[0:bn], f1_ref[0],
                  preferred_element_type=jnp.float32)
    for k in range(1, 12):
        hid = hid + jnp.dot(feat[k * bn:(k + 1) * bn], f1_ref[k],
                            preferred_element_type=jnp.float32)
    hid = jnp.maximum(hid + f1b_ref[...], 0.0).astype(jnp.bfloat16)
    logits = jnp.dot(hid, f2_ref[...],
                     preferred_element_type=jnp.float32) + f2b_ref[...]
    m = jnp.max(logits, axis=-1, keepdims=True)
    shifted = logits - m
    lse = jnp.log(jnp.sum(jnp.exp(shifted), axis=-1, keepdims=True))
    o_ref[...] = (shifted - lse).astype(o_ref.dtype)


def kernel(c1_w, c1_b, c2_w, c2_b, f1_w, f1_b, f2_w, f2_b, x):
    N = x.shape[0]
    xr = x.reshape(N, 28, 28)
    xe = xr[:, 0::2, :].transpose(1, 0, 2)
    xo = xr[:, 1::2, :].transpose(1, 0, 2)

    E1 = jnp.stack([jnp.eye(28, 26, k=-t, dtype=jnp.float32)
                    for t in range(3)])
    B1 = jnp.einsum('twv,dtc->dwvc', E1, c1_w.reshape(3, 3, 32))
    B1 = B1.reshape(84, 26, 32)
    pos = jnp.arange(48)
    wi_of_pos = 4 * (pos // 8) + pos % 8
    valid = (wi_of_pos <= 25).astype(jnp.float32)
    B1x = B1[:, jnp.clip(wi_of_pos, 0, 25), :] * valid[None, :, None]
    B1x = B1x.reshape(84, 1536)
    b1row = jnp.tile(c1_b, (1, 48))
    B1x = jnp.concatenate([B1x, b1row], axis=0).astype(jnp.bfloat16)

    pl_i = jnp.arange(8)[:, None, None, None]
    wpl_i = jnp.arange(2)[None, :, None, None]
    wpar_i = jnp.arange(2)[None, None, :, None]
    dw_i = jnp.arange(3)[None, None, None, :]
    D = (pl_i == 2 * wpl_i + wpar_i + dw_i).astype(jnp.float32)
    w2r = c2_w.reshape(3, 3, 32, 64)
    B2c = jnp.einsum('pwqd,hdcu->hpcqwu', D, w2r)
    B2c = B2c.reshape(3, 256, 256)
    B2core = B2c.reshape(768, 256)
    rmask = (jnp.arange(256) < 192).astype(jnp.float32)
    rmask = jnp.tile(rmask, (3,))[:, None]
    B2x = jnp.stack([B2core] * 5 + [B2core * rmask], axis=0)
    B2x = B2x.astype(jnp.bfloat16)

    b2t = jnp.tile(c2_b, (1, 12))
    f1 = f1_w.astype(jnp.bfloat16).reshape(12, 768, 128)
    f2 = f2_w.astype(jnp.bfloat16)

    bn = 64
    grid = (N // bn,)
    return pl.pallas_call(
        _fused_net_kernel,
        out_shape=jax.ShapeDtypeStruct((N, 10), x.dtype),
        grid=grid,
        in_specs=[
            pl.BlockSpec((14, bn, 28), lambda n: (0, n, 0)),
            pl.BlockSpec((14, bn, 28), lambda n: (0, n, 0)),
            pl.BlockSpec((85, 1536), lambda n: (0, 0)),
            pl.BlockSpec((6, 768, 256), lambda n: (0, 0, 0)),
            pl.BlockSpec((1, 768), lambda n: (0, 0)),
            pl.BlockSpec((12, 768, 128), lambda n: (0, 0, 0)),
            pl.BlockSpec((1, 128), lambda n: (0, 0)),
            pl.BlockSpec((128, 10), lambda n: (0, 0)),
            pl.BlockSpec((1, 10), lambda n: (0, 0)),
        ],
        out_specs=pl.BlockSpec((bn, 10), lambda n: (n, 0)),
        compiler_params=pltpu.CompilerParams(
            dimension_semantics=("parallel",),
            vmem_limit_bytes=64 * 1024 * 1024),
    )(xe, xo, B1x, B2x, b2t, f1, f1_b, f2, f2_b)


# speedup vs baseline: 21.0455x; 1.0401x over previous
import jax
import jax.numpy as jnp
from jax.experimental import pallas as pl
from jax.experimental.pallas import tpu as pltpu


def _fused_net_kernel(xe_ref, xo_ref, B1_ref, B2_ref, b2_ref,
                      f1_ref, f1b_ref, f2_ref, f2b_ref, o_ref):
    bn = xe_ref.shape[1]

    xe = xe_ref[...]
    xo = xo_ref[...]
    ones = jnp.ones((13, bn, 1), jnp.float32)
    xce = jnp.concatenate([xe[0:13], xo[0:13], xe[1:14], ones],
                          axis=-1).reshape(13 * bn, 85).astype(jnp.bfloat16)
    xco = jnp.concatenate([xo[0:13], xe[1:14], xo[1:14], ones],
                          axis=-1).reshape(13 * bn, 85).astype(jnp.bfloat16)
    h1e = jnp.maximum(jnp.dot(xce, B1_ref[...],
                              preferred_element_type=jnp.float32),
                      0.0).astype(jnp.bfloat16)
    h1o = jnp.maximum(jnp.dot(xco, B1_ref[...],
                              preferred_element_type=jnp.float32),
                      0.0).astype(jnp.bfloat16)

    slabs = {}
    for p in range(2):
        for dh in range(3):
            src = h1o if (p + dh) % 2 else h1e
            s = (p + dh) // 2
            slabs[(p, dh)] = src[s * bn:(s + 12) * bn]

    groups = []
    for g in range(6):
        lhs = jnp.concatenate(
            [jnp.concatenate([slabs[(p, dh)][:, 256 * g:256 * g + 256]
                              for dh in range(3)], axis=1)
             for p in range(2)], axis=0)
        a = jnp.dot(lhs, B2_ref[g], preferred_element_type=jnp.float32)
        aw = jnp.maximum(a[:, 0:128], a[:, 128:256])
        groups.append(jnp.maximum(aw[0:12 * bn], aw[12 * bn:]))
    pooled = jnp.concatenate(groups, axis=-1)
    feat = jnp.maximum(pooled + b2_ref[...], 0.0).astype(jnp.bfloat16)

    hid = jnp.dot(feat[0:bn], f1_ref[0],
                  preferred_element_type=jnp.float32)
    for k in range(1, 12):
        hid = hid + jnp.dot(feat[k * bn:(k + 1) * bn], f1_ref[k],
                            preferred_element_type=jnp.float32)
    hid = jnp.maximum(hid + f1b_ref[...], 0.0).astype(jnp.bfloat16)
    logits = jnp.dot(hid, f2_ref[...],
                     preferred_element_type=jnp.float32) + f2b_ref[...]
    m = jnp.max(logits, axis=-1, keepdims=True)
    shifted = logits - m
    lse = jnp.log(jnp.sum(jnp.exp(shifted), axis=-1, keepdims=True))
    o_ref[...] = (shifted - lse).astype(o_ref.dtype)


def kernel(c1_w, c1_b, c2_w, c2_b, f1_w, f1_b, f2_w, f2_b, x):
    N = x.shape[0]
    xr = x.reshape(N, 28, 28)
    xe = xr[:, 0::2, :].transpose(1, 0, 2)
    xo = xr[:, 1::2, :].transpose(1, 0, 2)

    E1 = jnp.stack([jnp.eye(28, 26, k=-t, dtype=jnp.float32)
                    for t in range(3)])
    B1 = jnp.einsum('twv,dtc->dwvc', E1, c1_w.reshape(3, 3, 32))
    B1 = B1.reshape(84, 26, 32)
    pos = jnp.arange(48)
    wi_of_pos = 4 * (pos // 8) + pos % 8
    valid = (wi_of_pos <= 25).astype(jnp.float32)
    B1x = B1[:, jnp.clip(wi_of_pos, 0, 25), :] * valid[None, :, None]
    B1x = B1x.reshape(84, 1536)
    b1row = jnp.tile(c1_b, (1, 48))
    B1x = jnp.concatenate([B1x, b1row], axis=0).astype(jnp.bfloat16)

    pl_i = jnp.arange(8)[:, None, None, None]
    wpl_i = jnp.arange(2)[None, :, None, None]
    wpar_i = jnp.arange(2)[None, None, :, None]
    dw_i = jnp.arange(3)[None, None, None, :]
    D = (pl_i == 2 * wpl_i + wpar_i + dw_i).astype(jnp.float32)
    w2r = c2_w.reshape(3, 3, 32, 64)
    B2c = jnp.einsum('pwqd,hdcu->hpcqwu', D, w2r)
    B2c = B2c.reshape(3, 256, 256)
    B2core = B2c.reshape(768, 256)
    rmask = (jnp.arange(256) < 192).astype(jnp.float32)
    rmask = jnp.tile(rmask, (3,))[:, None]
    B2x = jnp.stack([B2core] * 5 + [B2core * rmask], axis=0)
    B2x = B2x.astype(jnp.bfloat16)

    b2t = jnp.tile(c2_b, (1, 12))
    f1 = f1_w.astype(jnp.bfloat16).reshape(12, 768, 128)
    f2 = f2_w.astype(jnp.bfloat16)

    bn = 128
    grid = (N // bn,)
    return pl.pallas_call(
        _fused_net_kernel,
        out_shape=jax.ShapeDtypeStruct((N, 10), x.dtype),
        grid=grid,
        in_specs=[
            pl.BlockSpec((14, bn, 28), lambda n: (0, n, 0)),
            pl.BlockSpec((14, bn, 28), lambda n: (0, n, 0)),
            pl.BlockSpec((85, 1536), lambda n: (0, 0)),
            pl.BlockSpec((6, 768, 256), lambda n: (0, 0, 0)),
            pl.BlockSpec((1, 768), lambda n: (0, 0)),
            pl.BlockSpec((12, 768, 128), lambda n: (0, 0, 0)),
            pl.BlockSpec((1, 128), lambda n: (0, 0)),
            pl.BlockSpec((128, 10), lambda n: (0, 0)),
            pl.BlockSpec((1, 10), lambda n: (0, 0)),
        ],
        out_specs=pl.BlockSpec((bn, 10), lambda n: (n, 0)),
        compiler_params=pltpu.CompilerParams(
            dimension_semantics=("parallel",),
            vmem_limit_bytes=64 * 1024 * 1024),
    )(xe, xo, B1x, B2x, b2t, f1, f1_b, f2, f2_b)
